# baseline (speedup 1.0000x reference)
import functools

import jax
import jax.numpy as jnp
from jax import lax
from jax.experimental import pallas as pl
from jax.experimental.pallas import tpu as pltpu
from jax.experimental.pallas import tpu_sc as plsc

N = 50000
V = 128
G = 256
EPS = 1e-05
BLK = 1000
NB = N // BLK

NC = 2
NS = 16
CHUNK = 64
FULL = N // CHUNK
TAIL = N - FULL * CHUNK
GPAD = G + 8
MAGIC = 0x5F3759DF


def _node_norms(pv, flat_v, nrm_v):
    def body(n, _):
        for g in range(V // 16):
            sl = pl.ds(g * 16, 16)
            x = pv[n, 0, sl]
            y = pv[n, 1, sl]
            z = pv[n, 2, sl]
            flat_v[3 * n, sl] = x
            flat_v[3 * n + 1, sl] = y
            flat_v[3 * n + 2, sl] = z
            s2 = jnp.maximum(x * x + y * y + z * z, 1e-30)
            i = lax.bitcast_convert_type(s2, jnp.int32)
            r = lax.bitcast_convert_type(MAGIC - (i >> 1), jnp.float32)
            for _ in range(3):
                r = r * (1.5 - 0.5 * s2 * r * r)
            nrm_v[n, sl] = s2 * r
        return 0
    lax.fori_loop(0, CHUNK, body, 0)


def _sc_seg_body(pos_hbm, batch_hbm, pos3_hbm, seg_hbm, cnt_hbm,
                 pv, flat_v, nrm_v, ones_v, idx_v, zer_v, seg_sh, cnt_sh):
    cid = lax.axis_index("c")
    sid = lax.axis_index("s")
    gid = cid * NS + sid

    def _zrow(r, _):
        for g in range(V // 16):
            zer_v[r, pl.ds(g * 16, 16)] = jnp.zeros((16,), jnp.float32)
        return 0

    def _orow(r, _):
        for g in range(V // 16):
            ones_v[r, pl.ds(g * 16, 16)] = jnp.ones((16,), jnp.float32)
        return 0

    lax.fori_loop(0, 16, _zrow, 0)
    lax.fori_loop(0, CHUNK, _orow, 0)
    pltpu.sync_copy(zer_v, seg_sh.at[pl.ds(sid * 16, 16)])
    pltpu.sync_copy(zer_v, cnt_sh.at[pl.ds(sid * 16, 16)])

    @pl.when(sid == 0)
    def _():
        pltpu.sync_copy(zer_v.at[pl.ds(0, GPAD - G)],
                        seg_sh.at[pl.ds(G, GPAD - G)])
        pltpu.sync_copy(zer_v.at[pl.ds(0, GPAD - G)],
                        cnt_sh.at[pl.ds(G, GPAD - G)])

    plsc.subcore_barrier()

    for k in range(25):
        c = gid + 32 * k

        @pl.when(c < FULL)
        def _():
            base = c * CHUNK
            pltpu.sync_copy(batch_hbm.at[pl.ds(base, CHUNK)], idx_v)
            pltpu.sync_copy(pos_hbm.at[pl.ds(base, CHUNK)], pv)
            _node_norms(pv, flat_v, nrm_v)
            pltpu.sync_copy(flat_v, pos3_hbm.at[pl.ds(3 * base, 3 * CHUNK)])
            pltpu.sync_copy(nrm_v, seg_sh.at[idx_v], add=True)
            pltpu.sync_copy(ones_v, cnt_sh.at[idx_v], add=True)

        @pl.when(c == FULL)
        def _():
            for m in range(CHUNK // 16):
                idx_v[pl.ds(m * 16, 16)] = jnp.full((16,), G, jnp.int32)
            base = FULL * CHUNK
            pltpu.sync_copy(batch_hbm.at[pl.ds(base, TAIL)],
                            idx_v.at[pl.ds(0, TAIL)])
            pltpu.sync_copy(pos_hbm.at[pl.ds(base, TAIL)],
                            pv.at[pl.ds(0, TAIL)])
            _node_norms(pv, flat_v, nrm_v)
            pltpu.sync_copy(flat_v.at[pl.ds(0, 3 * TAIL)],
                            pos3_hbm.at[pl.ds(3 * base, 3 * TAIL)])
            pltpu.sync_copy(nrm_v, seg_sh.at[idx_v], add=True)
            pltpu.sync_copy(ones_v, cnt_sh.at[idx_v], add=True)

    plsc.subcore_barrier()

    @pl.when(sid == 0)
    def _():
        pltpu.sync_copy(seg_sh.at[pl.ds(0, G)], seg_hbm.at[cid])
        pltpu.sync_copy(cnt_sh.at[pl.ds(0, G)], cnt_hbm.at[cid])


@functools.lru_cache(maxsize=1)
def _make_sc_seg():
    mesh = plsc.VectorSubcoreMesh(core_axis_name="c", subcore_axis_name="s")
    return pl.kernel(
        _sc_seg_body,
        out_type=[
            jax.ShapeDtypeStruct((3 * N, V), jnp.float32),
            jax.ShapeDtypeStruct((NC, G, V), jnp.float32),
            jax.ShapeDtypeStruct((NC, G, V), jnp.float32),
        ],
        mesh=mesh,
        scratch_types=[
            pltpu.VMEM((CHUNK, 3, V), jnp.float32),
            pltpu.VMEM((3 * CHUNK, V), jnp.float32),
            pltpu.VMEM((CHUNK, V), jnp.float32),
            pltpu.VMEM((CHUNK, V), jnp.float32),
            pltpu.VMEM((CHUNK,), jnp.int32),
            pltpu.VMEM((16, V), jnp.float32),
            pltpu.VMEM_SHARED((GPAD, V), jnp.float32),
            pltpu.VMEM_SHARED((GPAD, V), jnp.float32),
        ],
    )


def _pass2_kernel(pos3_ref, batch3_ref, seg_ref, cnt_ref, w_ref, out_ref):
    x = pos3_ref[...]
    b = batch3_ref[0, 0, :]
    cnt = jnp.maximum(cnt_ref[0] + cnt_ref[1], 1.0)
    seg = seg_ref[0] + seg_ref[1]
    mean = seg / cnt
    oh = (b[:, None] == jax.lax.broadcasted_iota(jnp.int32, (3 * BLK, G), 1)
          ).astype(jnp.float32)
    gm = jnp.dot(oh, mean, preferred_element_type=jnp.float32)
    w = w_ref[0, 0, :]
    scale = w[None, :] / (gm + EPS)
    out_ref[...] = x * scale


def kernel(pos, weight, batch):
    b32 = batch.astype(jnp.int32)
    b3 = jnp.broadcast_to(b32[:, None], (N, 3)).reshape(NB, 1, 3 * BLK)

    pos3, seg, cnt = _make_sc_seg()(pos, b32)

    out3 = pl.pallas_call(
        _pass2_kernel,
        grid=(NB,),
        in_specs=[
            pl.BlockSpec((3 * BLK, V), lambda i: (i, 0)),
            pl.BlockSpec((1, 1, 3 * BLK), lambda i: (i, 0, 0)),
            pl.BlockSpec((NC, G, V), lambda i: (0, 0, 0)),
            pl.BlockSpec((NC, G, V), lambda i: (0, 0, 0)),
            pl.BlockSpec((1, 1, V), lambda i: (0, 0, 0)),
        ],
        out_specs=pl.BlockSpec((3 * BLK, V), lambda i: (i, 0)),
        out_shape=jax.ShapeDtypeStruct((3 * N, V), jnp.float32),
    )(pos3, b3, seg, cnt, weight)

    return out3.reshape(N, 3, V)


# speedup vs baseline: 1.5190x; 1.5190x over previous
import functools

import jax
import jax.numpy as jnp
from jax import lax
from jax.experimental import pallas as pl
from jax.experimental.pallas import tpu as pltpu
from jax.experimental.pallas import tpu_sc as plsc

N = 50000
V = 128
G = 256
EPS = 1e-05
BLK = 1000
NB = N // BLK

NC = 2
NS = 16
CHUNK = 128
FULL = N // CHUNK
TAIL = N - FULL * CHUNK
NCHUNK = FULL + 1
GPAD = G + 8


def _pass1_kernel(pos_ref, batch_ref, nrm_ref, cnt_ref):
    i = pl.program_id(0)
    x = pos_ref[...]
    x0 = x[:, 0, :]
    x1 = x[:, 1, :]
    x2 = x[:, 2, :]
    nrm_ref[...] = jnp.sqrt(x0 * x0 + x1 * x1 + x2 * x2)
    b = batch_ref[0, 0, :]
    oh = (jax.lax.broadcasted_iota(jnp.int32, (G, BLK), 0)
          == b[None, :]).astype(jnp.float32)
    pcnt = jnp.sum(oh, axis=1)[None, :]

    @pl.when(i == 0)
    def _():
        cnt_ref[...] = jnp.zeros_like(cnt_ref)

    cnt_ref[...] += pcnt


def _sc_seg_body(norm_hbm, batch_hbm, seg_hbm,
                 nrm_v, idx_v, zer_v, seg_sh):
    cid = lax.axis_index("c")
    sid = lax.axis_index("s")
    gid = cid * NS + sid

    def _zrow(r, _):
        for g in range(V // 16):
            zer_v[r, pl.ds(g * 16, 16)] = jnp.zeros((16,), jnp.float32)
        return 0
    lax.fori_loop(0, 16, _zrow, 0)
    pltpu.sync_copy(zer_v, seg_sh.at[pl.ds(sid * 16, 16)])

    @pl.when(sid == 0)
    def _():
        pltpu.sync_copy(zer_v.at[pl.ds(0, GPAD - G)],
                        seg_sh.at[pl.ds(G, GPAD - G)])

    plsc.subcore_barrier()

    for k in range(13):
        c = gid + 32 * k

        @pl.when(c < FULL)
        def _():
            base = c * CHUNK
            pltpu.sync_copy(batch_hbm.at[pl.ds(base, CHUNK)], idx_v)
            pltpu.sync_copy(norm_hbm.at[pl.ds(base, CHUNK)], nrm_v)
            pltpu.sync_copy(nrm_v, seg_sh.at[idx_v], add=True)

        @pl.when(c == FULL)
        def _():
            for m in range(CHUNK // 16):
                idx_v[pl.ds(m * 16, 16)] = jnp.full((16,), G, jnp.int32)
            pltpu.sync_copy(batch_hbm.at[pl.ds(FULL * CHUNK, TAIL)],
                            idx_v.at[pl.ds(0, TAIL)])
            pltpu.sync_copy(norm_hbm.at[pl.ds(FULL * CHUNK, TAIL)],
                            nrm_v.at[pl.ds(0, TAIL)])
            pltpu.sync_copy(nrm_v, seg_sh.at[idx_v], add=True)

    plsc.subcore_barrier()

    @pl.when(sid == 0)
    def _():
        pltpu.sync_copy(seg_sh.at[pl.ds(0, G)], seg_hbm.at[cid])


@functools.lru_cache(maxsize=1)
def _make_sc_seg():
    mesh = plsc.VectorSubcoreMesh(core_axis_name="c", subcore_axis_name="s")
    return pl.kernel(
        _sc_seg_body,
        out_type=jax.ShapeDtypeStruct((NC, G, V), jnp.float32),
        mesh=mesh,
        scratch_types=[
            pltpu.VMEM((CHUNK, V), jnp.float32),
            pltpu.VMEM((CHUNK,), jnp.int32),
            pltpu.VMEM((16, V), jnp.float32),
            pltpu.VMEM_SHARED((GPAD, V), jnp.float32),
        ],
    )


def _pass2_kernel(posf_ref, batch_ref, seg_ref, cnt_ref, w_ref, out_ref):
    x = posf_ref[...]
    b = batch_ref[0, 0, :]
    cnt = jnp.maximum(cnt_ref[0, :], 1.0)
    seg = seg_ref[0] + seg_ref[1]
    mean = seg / cnt[:, None]
    oh = (b[:, None] == jax.lax.broadcasted_iota(jnp.int32, (BLK, G), 1)
          ).astype(jnp.float32)
    gm = jnp.dot(oh, mean, preferred_element_type=jnp.float32)
    w = w_ref[0, 0, :]
    scale = w[None, :] / (gm + EPS)
    out_ref[:, :V] = x[:, :V] * scale
    out_ref[:, V:2 * V] = x[:, V:2 * V] * scale
    out_ref[:, 2 * V:] = x[:, 2 * V:] * scale


def kernel(pos, weight, batch):
    posf = pos.reshape(N, 3 * V)
    b32 = batch.astype(jnp.int32)
    b3 = b32.reshape(NB, 1, BLK)

    nrm, cnt = pl.pallas_call(
        _pass1_kernel,
        grid=(NB,),
        in_specs=[
            pl.BlockSpec((BLK, 3, V), lambda i: (i, 0, 0)),
            pl.BlockSpec((1, 1, BLK), lambda i: (i, 0, 0)),
        ],
        out_specs=[
            pl.BlockSpec((BLK, V), lambda i: (i, 0)),
            pl.BlockSpec((1, G), lambda i: (0, 0)),
        ],
        out_shape=[
            jax.ShapeDtypeStruct((N, V), jnp.float32),
            jax.ShapeDtypeStruct((1, G), jnp.float32),
        ],
    )(pos, b3)

    seg = _make_sc_seg()(nrm, b32)

    out = pl.pallas_call(
        _pass2_kernel,
        grid=(NB,),
        in_specs=[
            pl.BlockSpec((BLK, 3 * V), lambda i: (i, 0)),
            pl.BlockSpec((1, 1, BLK), lambda i: (i, 0, 0)),
            pl.BlockSpec((NC, G, V), lambda i: (0, 0, 0)),
            pl.BlockSpec((1, G), lambda i: (0, 0)),
            pl.BlockSpec((1, 1, V), lambda i: (0, 0, 0)),
        ],
        out_specs=pl.BlockSpec((BLK, 3 * V), lambda i: (i, 0)),
        out_shape=jax.ShapeDtypeStruct((N, 3 * V), jnp.float32),
    )(posf, b3, seg, cnt, weight)

    return out.reshape(N, 3, V)


# speedup vs baseline: 2.1687x; 1.4277x over previous
import functools

import jax
import jax.numpy as jnp
from jax import lax
from jax.experimental import pallas as pl
from jax.experimental.pallas import tpu as pltpu
from jax.experimental.pallas import tpu_sc as plsc

N = 50000
V = 128
G = 256
EPS = 1e-05
BLK = 2000
NB = N // BLK

NC = 2
NS = 16
CHUNK = 128
FULL = N // CHUNK
TAIL = N - FULL * CHUNK
NCHUNK = FULL + 1
GPAD = G + 8


def _pass1_kernel(posf_ref, batch_ref, nrm_ref, cnt_ref):
    i = pl.program_id(0)
    x = posf_ref[...]
    x0 = x[:, :V]
    x1 = x[:, V:2 * V]
    x2 = x[:, 2 * V:]
    nrm_ref[...] = jnp.sqrt(x0 * x0 + x1 * x1 + x2 * x2)
    b = batch_ref[0, 0, :]
    oh = (jax.lax.broadcasted_iota(jnp.int32, (G, BLK), 0)
          == b[None, :]).astype(jnp.float32)
    pcnt = jnp.sum(oh, axis=1)[None, :]

    @pl.when(i == 0)
    def _():
        cnt_ref[...] = jnp.zeros_like(cnt_ref)

    cnt_ref[...] += pcnt


def _sc_seg_body(norm_hbm, batch_hbm, seg_hbm,
                 nrm_v, idx_v, zer_v, seg_sh):
    cid = lax.axis_index("c")
    sid = lax.axis_index("s")
    gid = cid * NS + sid

    def _zrow(r, _):
        for g in range(V // 16):
            zer_v[r, pl.ds(g * 16, 16)] = jnp.zeros((16,), jnp.float32)
        return 0
    lax.fori_loop(0, 16, _zrow, 0)
    pltpu.sync_copy(zer_v, seg_sh.at[pl.ds(sid * 16, 16)])

    @pl.when(sid == 0)
    def _():
        pltpu.sync_copy(zer_v.at[pl.ds(0, GPAD - G)],
                        seg_sh.at[pl.ds(G, GPAD - G)])

    plsc.subcore_barrier()

    for k in range(13):
        c = gid + 32 * k

        @pl.when(c < FULL)
        def _():
            base = c * CHUNK
            pltpu.sync_copy(batch_hbm.at[pl.ds(base, CHUNK)], idx_v)
            pltpu.sync_copy(norm_hbm.at[pl.ds(base, CHUNK)], nrm_v)
            pltpu.sync_copy(nrm_v, seg_sh.at[idx_v], add=True)

        @pl.when(c == FULL)
        def _():
            for m in range(CHUNK // 16):
                idx_v[pl.ds(m * 16, 16)] = jnp.full((16,), G, jnp.int32)
            pltpu.sync_copy(batch_hbm.at[pl.ds(FULL * CHUNK, TAIL)],
                            idx_v.at[pl.ds(0, TAIL)])
            pltpu.sync_copy(norm_hbm.at[pl.ds(FULL * CHUNK, TAIL)],
                            nrm_v.at[pl.ds(0, TAIL)])
            pltpu.sync_copy(nrm_v, seg_sh.at[idx_v], add=True)

    plsc.subcore_barrier()

    @pl.when(sid == 0)
    def _():
        pltpu.sync_copy(seg_sh.at[pl.ds(0, G)], seg_hbm.at[cid])


@functools.lru_cache(maxsize=1)
def _make_sc_seg():
    mesh = plsc.VectorSubcoreMesh(core_axis_name="c", subcore_axis_name="s")
    return pl.kernel(
        _sc_seg_body,
        out_type=jax.ShapeDtypeStruct((NC, G, V), jnp.float32),
        mesh=mesh,
        scratch_types=[
            pltpu.VMEM((CHUNK, V), jnp.float32),
            pltpu.VMEM((CHUNK,), jnp.int32),
            pltpu.VMEM((16, V), jnp.float32),
            pltpu.VMEM_SHARED((GPAD, V), jnp.float32),
        ],
    )


def _pass2_kernel(posf_ref, batch_ref, seg_ref, cnt_ref, w_ref, out_ref):
    x = posf_ref[...]
    b = batch_ref[0, 0, :]
    cnt = jnp.maximum(cnt_ref[0, :], 1.0)
    seg = seg_ref[0] + seg_ref[1]
    mean = seg / cnt[:, None]
    oh = (b[:, None] == jax.lax.broadcasted_iota(jnp.int32, (BLK, G), 1)
          ).astype(jnp.float32)
    gm = jnp.dot(oh, mean, preferred_element_type=jnp.float32)
    w = w_ref[0, 0, :]
    scale = w[None, :] / (gm + EPS)
    out_ref[:, :V] = x[:, :V] * scale
    out_ref[:, V:2 * V] = x[:, V:2 * V] * scale
    out_ref[:, 2 * V:] = x[:, 2 * V:] * scale


def kernel(pos, weight, batch):
    posf = pos.reshape(N, 3 * V)
    b32 = batch.astype(jnp.int32)
    b3 = b32.reshape(NB, 1, BLK)

    nrm, cnt = pl.pallas_call(
        _pass1_kernel,
        grid=(NB,),
        in_specs=[
            pl.BlockSpec((BLK, 3 * V), lambda i: (i, 0)),
            pl.BlockSpec((1, 1, BLK), lambda i: (i, 0, 0)),
        ],
        out_specs=[
            pl.BlockSpec((BLK, V), lambda i: (i, 0)),
            pl.BlockSpec((1, G), lambda i: (0, 0)),
        ],
        out_shape=[
            jax.ShapeDtypeStruct((N, V), jnp.float32),
            jax.ShapeDtypeStruct((1, G), jnp.float32),
        ],
    )(posf, b3)

    seg = _make_sc_seg()(nrm, b32)

    out = pl.pallas_call(
        _pass2_kernel,
        grid=(NB,),
        in_specs=[
            pl.BlockSpec((BLK, 3 * V), lambda i: (i, 0)),
            pl.BlockSpec((1, 1, BLK), lambda i: (i, 0, 0)),
            pl.BlockSpec((NC, G, V), lambda i: (0, 0, 0)),
            pl.BlockSpec((1, G), lambda i: (0, 0)),
            pl.BlockSpec((1, 1, V), lambda i: (0, 0, 0)),
        ],
        out_specs=pl.BlockSpec((BLK, 3 * V), lambda i: (i, 0)),
        out_shape=jax.ShapeDtypeStruct((N, 3 * V), jnp.float32),
    )(posf, b3, seg, cnt, weight)

    return out.reshape(N, 3, V)


# speedup vs baseline: 2.2310x; 1.0287x over previous
import functools

import jax
import jax.numpy as jnp
from jax import lax
from jax.experimental import pallas as pl
from jax.experimental.pallas import tpu as pltpu
from jax.experimental.pallas import tpu_sc as plsc

N = 50000
V = 128
G = 256
EPS = 1e-05
BLK = 5000
NB = N // BLK

NC = 2
NS = 16
CHUNK = 128
FULL = N // CHUNK
TAIL = N - FULL * CHUNK
NCHUNK = FULL + 1
GPAD = G + 8


def _pass1_kernel(posf_ref, batch_ref, nrm_ref, cnt_ref):
    i = pl.program_id(0)
    x = posf_ref[...]
    x0 = x[:, :V]
    x1 = x[:, V:2 * V]
    x2 = x[:, 2 * V:]
    nrm_ref[...] = jnp.sqrt(x0 * x0 + x1 * x1 + x2 * x2)
    b = batch_ref[0, 0, :]
    oh = (jax.lax.broadcasted_iota(jnp.int32, (G, BLK), 0)
          == b[None, :]).astype(jnp.float32)
    pcnt = jnp.sum(oh, axis=1)[None, :]

    @pl.when(i == 0)
    def _():
        cnt_ref[...] = jnp.zeros_like(cnt_ref)

    cnt_ref[...] += pcnt


def _sc_seg_body(norm_hbm, batch_hbm, seg_hbm,
                 nrm_v, idx_v, zer_v, seg_sh):
    cid = lax.axis_index("c")
    sid = lax.axis_index("s")
    gid = cid * NS + sid

    def _zrow(r, _):
        for g in range(V // 16):
            zer_v[r, pl.ds(g * 16, 16)] = jnp.zeros((16,), jnp.float32)
        return 0
    lax.fori_loop(0, 16, _zrow, 0)
    pltpu.sync_copy(zer_v, seg_sh.at[pl.ds(sid * 16, 16)])

    @pl.when(sid == 0)
    def _():
        pltpu.sync_copy(zer_v.at[pl.ds(0, GPAD - G)],
                        seg_sh.at[pl.ds(G, GPAD - G)])

    plsc.subcore_barrier()

    for k in range(13):
        c = gid + 32 * k

        @pl.when(c < FULL)
        def _():
            base = c * CHUNK
            pltpu.sync_copy(batch_hbm.at[pl.ds(base, CHUNK)], idx_v)
            pltpu.sync_copy(norm_hbm.at[pl.ds(base, CHUNK)], nrm_v)
            pltpu.sync_copy(nrm_v, seg_sh.at[idx_v], add=True)

        @pl.when(c == FULL)
        def _():
            for m in range(CHUNK // 16):
                idx_v[pl.ds(m * 16, 16)] = jnp.full((16,), G, jnp.int32)
            pltpu.sync_copy(batch_hbm.at[pl.ds(FULL * CHUNK, TAIL)],
                            idx_v.at[pl.ds(0, TAIL)])
            pltpu.sync_copy(norm_hbm.at[pl.ds(FULL * CHUNK, TAIL)],
                            nrm_v.at[pl.ds(0, TAIL)])
            pltpu.sync_copy(nrm_v, seg_sh.at[idx_v], add=True)

    plsc.subcore_barrier()

    @pl.when(sid == 0)
    def _():
        pltpu.sync_copy(seg_sh.at[pl.ds(0, G)], seg_hbm.at[cid])


@functools.lru_cache(maxsize=1)
def _make_sc_seg():
    mesh = plsc.VectorSubcoreMesh(core_axis_name="c", subcore_axis_name="s")
    return pl.kernel(
        _sc_seg_body,
        out_type=jax.ShapeDtypeStruct((NC, G, V), jnp.float32),
        mesh=mesh,
        scratch_types=[
            pltpu.VMEM((CHUNK, V), jnp.float32),
            pltpu.VMEM((CHUNK,), jnp.int32),
            pltpu.VMEM((16, V), jnp.float32),
            pltpu.VMEM_SHARED((GPAD, V), jnp.float32),
        ],
    )


def _pass2_kernel(posf_ref, batch_ref, seg_ref, cnt_ref, w_ref, out_ref):
    x = posf_ref[...]
    b = batch_ref[0, 0, :]
    cnt = jnp.maximum(cnt_ref[0, :], 1.0)
    seg = seg_ref[0] + seg_ref[1]
    mean = seg / cnt[:, None]
    oh = (b[:, None] == jax.lax.broadcasted_iota(jnp.int32, (BLK, G), 1)
          ).astype(jnp.float32)
    gm = jnp.dot(oh, mean, preferred_element_type=jnp.float32)
    w = w_ref[0, 0, :]
    scale = w[None, :] / (gm + EPS)
    out_ref[:, :V] = x[:, :V] * scale
    out_ref[:, V:2 * V] = x[:, V:2 * V] * scale
    out_ref[:, 2 * V:] = x[:, 2 * V:] * scale


def kernel(pos, weight, batch):
    posf = pos.reshape(N, 3 * V)
    b32 = batch.astype(jnp.int32)
    b3 = b32.reshape(NB, 1, BLK)

    nrm, cnt = pl.pallas_call(
        _pass1_kernel,
        grid=(NB,),
        in_specs=[
            pl.BlockSpec((BLK, 3 * V), lambda i: (i, 0)),
            pl.BlockSpec((1, 1, BLK), lambda i: (i, 0, 0)),
        ],
        out_specs=[
            pl.BlockSpec((BLK, V), lambda i: (i, 0)),
            pl.BlockSpec((1, G), lambda i: (0, 0)),
        ],
        out_shape=[
            jax.ShapeDtypeStruct((N, V), jnp.float32),
            jax.ShapeDtypeStruct((1, G), jnp.float32),
        ],
    )(posf, b3)

    seg = _make_sc_seg()(nrm, b32)

    out = pl.pallas_call(
        _pass2_kernel,
        grid=(NB,),
        in_specs=[
            pl.BlockSpec((BLK, 3 * V), lambda i: (i, 0)),
            pl.BlockSpec((1, 1, BLK), lambda i: (i, 0, 0)),
            pl.BlockSpec((NC, G, V), lambda i: (0, 0, 0)),
            pl.BlockSpec((1, G), lambda i: (0, 0)),
            pl.BlockSpec((1, 1, V), lambda i: (0, 0, 0)),
        ],
        out_specs=pl.BlockSpec((BLK, 3 * V), lambda i: (i, 0)),
        out_shape=jax.ShapeDtypeStruct((N, 3 * V), jnp.float32),
    )(posf, b3, seg, cnt, weight)

    return out.reshape(N, 3, V)


# speedup vs baseline: 2.2319x; 1.0004x over previous
import functools

import jax
import jax.numpy as jnp
from jax import lax
from jax.experimental import pallas as pl
from jax.experimental.pallas import tpu as pltpu
from jax.experimental.pallas import tpu_sc as plsc

N = 50000
V = 128
G = 256
EPS = 1e-05
BLK = 5000
NB = N // BLK

NC = 2
NS = 16
CHUNK = 128
FULL = N // CHUNK
TAIL = N - FULL * CHUNK
GPAD = G + 8


def _pass1_kernel(posf_ref, batch_ref, nrm_ref, cnt_ref):
    i = pl.program_id(0)
    x = posf_ref[...]
    x0 = x[:, :V]
    x1 = x[:, V:2 * V]
    x2 = x[:, 2 * V:]
    nrm_ref[...] = jnp.sqrt(x0 * x0 + x1 * x1 + x2 * x2)
    b = batch_ref[0, 0, :]
    oh = (jax.lax.broadcasted_iota(jnp.int32, (G, BLK), 0)
          == b[None, :]).astype(jnp.float32)
    pcnt = jnp.sum(oh, axis=1)[None, :]

    @pl.when(i == 0)
    def _():
        cnt_ref[...] = jnp.zeros_like(cnt_ref)

    cnt_ref[...] += pcnt


def _sc_seg_body(norm_hbm, batch_hbm, seg_hbm,
                 nrm_v, idx_v, zer_v, seg_sh):
    cid = lax.axis_index("c")
    sid = lax.axis_index("s")
    gid = cid * NS + sid

    def _zrow(r, _):
        for g in range(V // 16):
            zer_v[r, pl.ds(g * 16, 16)] = jnp.zeros((16,), jnp.float32)
        return 0
    lax.fori_loop(0, 16, _zrow, 0)
    pltpu.sync_copy(zer_v, seg_sh.at[pl.ds(sid * 16, 16)])

    @pl.when(sid == 0)
    def _():
        pltpu.sync_copy(zer_v.at[pl.ds(0, GPAD - G)],
                        seg_sh.at[pl.ds(G, GPAD - G)])

    plsc.subcore_barrier()

    for k in range(13):
        c = gid + 32 * k

        @pl.when(c < FULL)
        def _():
            base = c * CHUNK
            pltpu.sync_copy(batch_hbm.at[pl.ds(base, CHUNK)], idx_v)
            pltpu.sync_copy(norm_hbm.at[pl.ds(base, CHUNK)], nrm_v)
            pltpu.sync_copy(nrm_v, seg_sh.at[idx_v], add=True)

        @pl.when(c == FULL)
        def _():
            for m in range(CHUNK // 16):
                idx_v[pl.ds(m * 16, 16)] = jnp.full((16,), G, jnp.int32)
            pltpu.sync_copy(batch_hbm.at[pl.ds(FULL * CHUNK, TAIL)],
                            idx_v.at[pl.ds(0, TAIL)])
            pltpu.sync_copy(norm_hbm.at[pl.ds(FULL * CHUNK, TAIL)],
                            nrm_v.at[pl.ds(0, TAIL)])
            pltpu.sync_copy(nrm_v, seg_sh.at[idx_v], add=True)

    plsc.subcore_barrier()

    @pl.when(sid == 0)
    def _():
        pltpu.sync_copy(seg_sh.at[pl.ds(0, G)], seg_hbm.at[cid])


@functools.lru_cache(maxsize=1)
def _make_sc_seg():
    mesh = plsc.VectorSubcoreMesh(core_axis_name="c", subcore_axis_name="s")
    return pl.kernel(
        _sc_seg_body,
        out_type=jax.ShapeDtypeStruct((NC, G, V), jnp.float32),
        mesh=mesh,
        scratch_types=[
            pltpu.VMEM((CHUNK, V), jnp.float32),
            pltpu.VMEM((CHUNK,), jnp.int32),
            pltpu.VMEM((16, V), jnp.float32),
            pltpu.VMEM_SHARED((GPAD, V), jnp.float32),
        ],
    )


def _pass2_kernel(posf_ref, batch_ref, seg_ref, cnt_ref, w_ref, out_ref):
    x = posf_ref[...]
    b = batch_ref[0, 0, :]
    cnt = jnp.maximum(cnt_ref[0, :], 1.0)
    seg = seg_ref[0] + seg_ref[1]
    mean = seg / cnt[:, None]
    oh = (b[:, None] == jax.lax.broadcasted_iota(jnp.int32, (BLK, G), 1)
          ).astype(jnp.float32)
    gm = jnp.dot(oh, mean, preferred_element_type=jnp.float32)
    w = w_ref[0, 0, :]
    scale = w[None, :] / (gm + EPS)
    out_ref[:, :V] = x[:, :V] * scale
    out_ref[:, V:2 * V] = x[:, V:2 * V] * scale
    out_ref[:, 2 * V:] = x[:, 2 * V:] * scale


def kernel(pos, weight, batch):
    posf = pos.reshape(N, 3 * V)
    b32 = batch.astype(jnp.int32)
    b3 = b32.reshape(NB, 1, BLK)

    nrm, cnt = pl.pallas_call(
        _pass1_kernel,
        grid=(NB,),
        in_specs=[
            pl.BlockSpec((BLK, 3 * V), lambda i: (i, 0)),
            pl.BlockSpec((1, 1, BLK), lambda i: (i, 0, 0)),
        ],
        out_specs=[
            pl.BlockSpec((BLK, V), lambda i: (i, 0)),
            pl.BlockSpec((1, G), lambda i: (0, 0)),
        ],
        out_shape=[
            jax.ShapeDtypeStruct((N, V), jnp.float32),
            jax.ShapeDtypeStruct((1, G), jnp.float32),
        ],
    )(posf, b3)

    seg = _make_sc_seg()(nrm, b32)

    out = pl.pallas_call(
        _pass2_kernel,
        grid=(NB,),
        in_specs=[
            pl.BlockSpec((BLK, 3 * V), lambda i: (i, 0)),
            pl.BlockSpec((1, 1, BLK), lambda i: (i, 0, 0)),
            pl.BlockSpec((NC, G, V), lambda i: (0, 0, 0)),
            pl.BlockSpec((1, G), lambda i: (0, 0)),
            pl.BlockSpec((1, 1, V), lambda i: (0, 0, 0)),
        ],
        out_specs=pl.BlockSpec((BLK, 3 * V), lambda i: (i, 0)),
        out_shape=jax.ShapeDtypeStruct((N, 3 * V), jnp.float32),
    )(posf, b3, seg, cnt, weight)

    return out.reshape(N, 3, V)
